# baseline (speedup 1.0000x reference)
import functools

import jax
import jax.numpy as jnp
from jax import lax
from jax.experimental import pallas as pl
from jax.experimental.pallas import tpu as pltpu
from jax.experimental.pallas import tpu_sc as plsc

N = 10000
D = 128
E = 320000
NC = 2
NS = 16
NW = NC * NS
CH = 128
NCHUNK = 80
NBUF = 2
IDXG = 8
EPAD = NW * NCHUNK * CH
ACC_ROWS = 10112
ROWS_PER_TILE = ACC_ROWS // NS
CNT_W = 16
BLK = 400
FEAT_SPLIT = (144, 16)
TREP = 32

_MESH = dict(core_axis_name="c", subcore_axis_name="s")


def _sc_features(xs, xm, xr, edges):
    zeros_f = jnp.zeros((CH, D), jnp.float32)

    C0, C1 = FEAT_SPLIT

    @functools.partial(
        pl.kernel,
        out_type=jax.ShapeDtypeStruct((5 * NC, ACC_ROWS, D), jnp.float32),
        mesh=plsc.VectorSubcoreMesh(**_MESH),
        scratch_types=[
            pltpu.VMEM_SHARED((ACC_ROWS, D), jnp.float32),
            pltpu.VMEM((2, IDXG, CH), jnp.int32),
            pltpu.VMEM((2, IDXG, CH), jnp.int32),
            pltpu.VMEM((NBUF, CH, D), jnp.float32),
            pltpu.SemaphoreType.DMA,
            pltpu.SemaphoreType.DMA,
        ],
    )
    def agg(xs_h, xm_h, xr_h,
            s0, d0, s1, d1, s2, d2, s3, d3, s4, d4, zf_h,
            sums_h,
            acc_f, sidx, didx, rows,
            sem0, sem1):
        cid = lax.axis_index("c")
        sid = lax.axis_index("s")
        base = sid * ROWS_PER_TILE
        crow = jnp.where(cid == 0, sid * C0, NS * C0 + sid * C1)
        ngrp = jnp.where(cid == 0, C0 // IDXG, C1 // IDXG)
        sems = [sem0, sem1]

        def wait_buf(tab, b):
            pltpu.make_async_copy(tab.at[pl.ds(0, CH)], rows.at[b],
                                  sems[b]).wait()

        rels = [(xs_h, s0, d0), (xs_h, s1, d1), (xm_h, s2, d2),
                (xm_h, s3, d3), (xr_h, s4, d4)]
        for r, (tab, se, de) in enumerate(rels):
            pltpu.sync_copy(zf_h, rows.at[0])
            for k in range(5):
                n = min(CH, ROWS_PER_TILE - k * CH)
                pltpu.sync_copy(rows.at[0, pl.ds(0, n)],
                                acc_f.at[pl.ds(base + k * CH, n)])
            plsc.subcore_barrier()

            pltpu.sync_copy(se.at[pl.ds(crow, IDXG)], sidx.at[0])
            pltpu.sync_copy(de.at[pl.ds(crow, IDXG)], didx.at[0])
            for b in range(NBUF):
                pltpu.async_copy(tab.at[sidx.at[0, b]], rows.at[b], sems[b])

            def group(g, carry):
                gn = lax.rem(g + 1, ngrp)
                pltpu.sync_copy(se.at[pl.ds(crow + gn * IDXG, IDXG)],
                                sidx.at[(g + 1) % 2])
                pltpu.sync_copy(de.at[pl.ds(crow + gn * IDXG, IDXG)],
                                didx.at[(g + 1) % 2])
                for jj in range(IDXG):
                    b = jj % NBUF
                    wait_buf(tab, b)
                    pltpu.sync_copy(rows.at[b],
                                    acc_f.at[didx.at[g % 2, jj]], add=True)
                    if jj + NBUF < IDXG:
                        pltpu.async_copy(tab.at[sidx.at[g % 2, jj + NBUF]],
                                         rows.at[b], sems[b])
                    else:
                        pltpu.async_copy(
                            tab.at[sidx.at[(g + 1) % 2, jj + NBUF - IDXG]],
                            rows.at[b], sems[b])
                return carry

            lax.fori_loop(0, ngrp, group, 0, unroll=False)
            for b in range(NBUF):
                wait_buf(tab, b)
            plsc.subcore_barrier()

            for k in range(5):
                n = min(CH, ROWS_PER_TILE - k * CH)
                pltpu.sync_copy(acc_f.at[pl.ds(base + k * CH, n)],
                                rows.at[0, pl.ds(0, n)])
                pltpu.sync_copy(rows.at[0, pl.ds(0, n)],
                                sums_h.at[r * NC + cid,
                                          pl.ds(base + k * CH, n)])

    flat = []
    for s, d in edges:
        flat += [s, d]
    return agg(xs, xm, xr, *flat, zeros_f)


def _sc_counts(edges):
    import numpy as np
    pat = np.zeros((5, CH, D), np.float32)
    for r in range(5):
        pat[r, :, 16 * r:16 * (r + 1)] = 1.0
    ones_pat = jnp.asarray(pat)
    zeros_f = jnp.zeros((CH, D), jnp.float32)

    @functools.partial(
        pl.kernel,
        out_type=jax.ShapeDtypeStruct((NC, ACC_ROWS, D), jnp.float32),
        mesh=plsc.VectorSubcoreMesh(**_MESH),
        scratch_types=[
            pltpu.VMEM_SHARED((ACC_ROWS, D), jnp.float32),
            pltpu.VMEM((IDXG, CH), jnp.int32),
            pltpu.VMEM((CH, D), jnp.float32),
            pltpu.VMEM((CH, D), jnp.float32),
        ],
    )
    def cnt(d0, d1, d2, d3, d4, ones_h, zf_h,
            cnts_h,
            acc, didx, ones_v, stage):
        cid = lax.axis_index("c")
        sid = lax.axis_index("s")
        base = sid * ROWS_PER_TILE
        crow = (cid * NS + sid) * NCHUNK

        pltpu.sync_copy(zf_h, stage)
        for k in range(5):
            n = min(CH, ROWS_PER_TILE - k * CH)
            pltpu.sync_copy(stage.at[pl.ds(0, n)],
                            acc.at[pl.ds(base + k * CH, n)])
        plsc.subcore_barrier()

        for r, de in enumerate([d0, d1, d2, d3, d4]):
            pltpu.sync_copy(ones_h.at[r], ones_v)

            def group(g, carry):
                pltpu.sync_copy(de.at[pl.ds(crow + g * IDXG, IDXG)], didx)
                for jj in range(IDXG):
                    pltpu.sync_copy(ones_v, acc.at[didx.at[jj]], add=True)
                return carry

            lax.fori_loop(0, NCHUNK // IDXG, group, 0)
        plsc.subcore_barrier()

        for k in range(5):
            n = min(CH, ROWS_PER_TILE - k * CH)
            pltpu.sync_copy(acc.at[pl.ds(base + k * CH, n)],
                            stage.at[pl.ds(0, n)])
            pltpu.sync_copy(stage.at[pl.ds(0, n)],
                            cnts_h.at[cid, pl.ds(base + k * CH, n)])

    return cnt(*[d for _, d in edges], ones_pat, zeros_f)


def _tc_body(sums_ref, cnts_ref, xj_ref, wl_ref, wr_ref, bl_ref, out_ref):
    acc = jnp.dot(xj_ref[...], wr_ref[...],
                  preferred_element_type=jnp.float32) + bl_ref[...]
    for r in range(5):
        s = sums_ref[2 * r] + sums_ref[2 * r + 1]
        c = cnts_ref[0, :, 16 * r] + cnts_ref[1, :, 16 * r]
        mean = s / jnp.maximum(c, 1.0)[:, None]
        acc = acc + jnp.dot(mean, wl_ref[r], preferred_element_type=jnp.float32)
    out_ref[...] = jnp.maximum(acc, 0.0)


def _tc_combine(sums, cnts, x_job, wl_t, wr_t, bl):
    return pl.pallas_call(
        _tc_body,
        grid=(N // BLK,),
        in_specs=[
            pl.BlockSpec((5 * NC, BLK, D), lambda i: (0, i, 0)),
            pl.BlockSpec((NC, BLK, D), lambda i: (0, i, 0)),
            pl.BlockSpec((BLK, D), lambda i: (i, 0)),
            pl.BlockSpec((5, D, D), lambda i: (0, 0, 0)),
            pl.BlockSpec((D, D), lambda i: (0, 0)),
            pl.BlockSpec((1, D), lambda i: (0, 0)),
        ],
        out_specs=pl.BlockSpec((BLK, D), lambda i: (i, 0)),
        out_shape=jax.ShapeDtypeStruct((N, D), jnp.float32),
    )(sums, cnts, x_job, wl_t, wr_t, bl)


def _prep_edges(ei):
    ei = ei.astype(jnp.int32)
    src = jnp.concatenate([ei[0], jnp.zeros((EPAD - E,), jnp.int32)])
    dst = jnp.concatenate([ei[1], jnp.full((EPAD - E,), N, jnp.int32)])
    src = src.reshape(NW * NCHUNK, CH)
    rep = (jnp.arange(NW * NCHUNK, dtype=jnp.int32) % TREP) * N
    return (src + rep[:, None], dst.reshape(NW * NCHUNK, CH))


def kernel(x_station, x_machine, x_robot, x_job,
           edge_can_load, edge_loaded, edge_will_execute, edge_execute,
           edge_hold,
           Wl_can_load, bl_can_load, Wr_can_load,
           Wl_loaded, bl_loaded, Wr_loaded,
           Wl_will_execute, bl_will_execute, Wr_will_execute,
           Wl_execute, bl_execute, Wr_execute,
           Wl_hold, bl_hold, Wr_hold):
    edges = [_prep_edges(e) for e in (edge_can_load, edge_loaded,
                                      edge_will_execute, edge_execute,
                                      edge_hold)]
    rep = (TREP, 1)
    sums = _sc_features(jnp.tile(x_station, rep), jnp.tile(x_machine, rep),
                        jnp.tile(x_robot, rep), edges)
    cnts = _sc_counts(edges)

    wl_t = jnp.stack([Wl_can_load.T, Wl_loaded.T, Wl_will_execute.T,
                      Wl_execute.T, Wl_hold.T])
    wr_t = (Wr_can_load + Wr_loaded + Wr_will_execute
            + Wr_execute + Wr_hold).T
    bl = (bl_can_load + bl_loaded + bl_will_execute
          + bl_execute + bl_hold).reshape(1, D)
    return _tc_combine(sums, cnts, x_job, wl_t, wr_t, bl)


# speedup vs baseline: 1.0364x; 1.0364x over previous
import functools

import jax
import jax.numpy as jnp
from jax import lax
from jax.experimental import pallas as pl
from jax.experimental.pallas import tpu as pltpu
from jax.experimental.pallas import tpu_sc as plsc

N = 10000
D = 128
E = 320000
NC = 2
NS = 16
NW = NC * NS
CH = 128
NCHUNK = 80
NBUF = 2
IDXG = 8
EPAD = NW * NCHUNK * CH
ACC_ROWS = 10112
ROWS_PER_TILE = ACC_ROWS // NS
CNT_W = 16
BLK = 400
FEAT_SPLIT = (136, 24)
TREP = 32

_MESH = dict(core_axis_name="c", subcore_axis_name="s")


def _sc_features(xs, xm, xr, edges):
    zeros_f = jnp.zeros((CH, D), jnp.float32)

    C0, C1 = FEAT_SPLIT

    @functools.partial(
        pl.kernel,
        out_type=jax.ShapeDtypeStruct((5 * NC, ACC_ROWS, D), jnp.float32),
        mesh=plsc.VectorSubcoreMesh(**_MESH),
        scratch_types=[
            pltpu.VMEM_SHARED((ACC_ROWS, D), jnp.float32),
            pltpu.VMEM((2, IDXG, CH), jnp.int32),
            pltpu.VMEM((2, IDXG, CH), jnp.int32),
            pltpu.VMEM((NBUF, CH, D), jnp.float32),
            pltpu.SemaphoreType.DMA,
            pltpu.SemaphoreType.DMA,
        ],
    )
    def agg(xs_h, xm_h, xr_h,
            s0, d0, s1, d1, s2, d2, s3, d3, s4, d4, zf_h,
            sums_h,
            acc_f, sidx, didx, rows,
            sem0, sem1):
        cid = lax.axis_index("c")
        sid = lax.axis_index("s")
        base = sid * ROWS_PER_TILE
        crow = jnp.where(cid == 0, sid * C0, NS * C0 + sid * C1)
        ngrp = jnp.where(cid == 0, C0 // IDXG, C1 // IDXG)
        sems = [sem0, sem1]

        def wait_buf(tab, b):
            pltpu.make_async_copy(tab.at[pl.ds(0, CH)], rows.at[b],
                                  sems[b]).wait()

        rels = [(xs_h, s0, d0), (xs_h, s1, d1), (xm_h, s2, d2),
                (xm_h, s3, d3), (xr_h, s4, d4)]
        for r, (tab, se, de) in enumerate(rels):
            pltpu.sync_copy(zf_h, rows.at[0])
            for k in range(5):
                n = min(CH, ROWS_PER_TILE - k * CH)
                pltpu.sync_copy(rows.at[0, pl.ds(0, n)],
                                acc_f.at[pl.ds(base + k * CH, n)])
            plsc.subcore_barrier()

            pltpu.sync_copy(se.at[pl.ds(crow, IDXG)], sidx.at[0])
            pltpu.sync_copy(de.at[pl.ds(crow, IDXG)], didx.at[0])
            for b in range(NBUF):
                pltpu.async_copy(tab.at[sidx.at[0, b]], rows.at[b], sems[b])

            def group(g, carry):
                gn = lax.rem(g + 1, ngrp)
                pltpu.sync_copy(se.at[pl.ds(crow + gn * IDXG, IDXG)],
                                sidx.at[(g + 1) % 2])
                pltpu.sync_copy(de.at[pl.ds(crow + gn * IDXG, IDXG)],
                                didx.at[(g + 1) % 2])
                for jj in range(IDXG):
                    b = jj % NBUF
                    wait_buf(tab, b)
                    pltpu.sync_copy(rows.at[b],
                                    acc_f.at[didx.at[g % 2, jj]], add=True)
                    if jj + NBUF < IDXG:
                        pltpu.async_copy(tab.at[sidx.at[g % 2, jj + NBUF]],
                                         rows.at[b], sems[b])
                    else:
                        pltpu.async_copy(
                            tab.at[sidx.at[(g + 1) % 2, jj + NBUF - IDXG]],
                            rows.at[b], sems[b])
                return carry

            lax.fori_loop(0, ngrp, group, 0, unroll=False)
            for b in range(NBUF):
                wait_buf(tab, b)
            plsc.subcore_barrier()

            for k in range(5):
                n = min(CH, ROWS_PER_TILE - k * CH)
                pltpu.sync_copy(acc_f.at[pl.ds(base + k * CH, n)],
                                rows.at[0, pl.ds(0, n)])
                pltpu.sync_copy(rows.at[0, pl.ds(0, n)],
                                sums_h.at[r * NC + cid,
                                          pl.ds(base + k * CH, n)])

    flat = []
    for s, d in edges:
        flat += [s, d]
    return agg(xs, xm, xr, *flat, zeros_f)


def _sc_counts(edges):
    import numpy as np
    pat = np.zeros((5, CH, D), np.float32)
    for r in range(5):
        pat[r, :, 16 * r:16 * (r + 1)] = 1.0
    ones_pat = jnp.asarray(pat)
    zeros_f = jnp.zeros((CH, D), jnp.float32)

    @functools.partial(
        pl.kernel,
        out_type=jax.ShapeDtypeStruct((NC, ACC_ROWS, D), jnp.float32),
        mesh=plsc.VectorSubcoreMesh(**_MESH),
        scratch_types=[
            pltpu.VMEM_SHARED((ACC_ROWS, D), jnp.float32),
            pltpu.VMEM((IDXG, CH), jnp.int32),
            pltpu.VMEM((CH, D), jnp.float32),
            pltpu.VMEM((CH, D), jnp.float32),
        ],
    )
    def cnt(d0, d1, d2, d3, d4, ones_h, zf_h,
            cnts_h,
            acc, didx, ones_v, stage):
        cid = lax.axis_index("c")
        sid = lax.axis_index("s")
        base = sid * ROWS_PER_TILE
        crow = (cid * NS + sid) * NCHUNK

        pltpu.sync_copy(zf_h, stage)
        for k in range(5):
            n = min(CH, ROWS_PER_TILE - k * CH)
            pltpu.sync_copy(stage.at[pl.ds(0, n)],
                            acc.at[pl.ds(base + k * CH, n)])
        plsc.subcore_barrier()

        for r, de in enumerate([d0, d1, d2, d3, d4]):
            pltpu.sync_copy(ones_h.at[r], ones_v)

            def group(g, carry):
                pltpu.sync_copy(de.at[pl.ds(crow + g * IDXG, IDXG)], didx)
                for jj in range(IDXG):
                    pltpu.sync_copy(ones_v, acc.at[didx.at[jj]], add=True)
                return carry

            lax.fori_loop(0, NCHUNK // IDXG, group, 0)
        plsc.subcore_barrier()

        for k in range(5):
            n = min(CH, ROWS_PER_TILE - k * CH)
            pltpu.sync_copy(acc.at[pl.ds(base + k * CH, n)],
                            stage.at[pl.ds(0, n)])
            pltpu.sync_copy(stage.at[pl.ds(0, n)],
                            cnts_h.at[cid, pl.ds(base + k * CH, n)])

    return cnt(*[d for _, d in edges], ones_pat, zeros_f)


def _tc_body(sums_ref, cnts_ref, xj_ref, wl_ref, wr_ref, bl_ref, out_ref):
    acc = jnp.dot(xj_ref[...], wr_ref[...],
                  preferred_element_type=jnp.float32) + bl_ref[...]
    for r in range(5):
        s = sums_ref[2 * r] + sums_ref[2 * r + 1]
        c = cnts_ref[0, :, 16 * r] + cnts_ref[1, :, 16 * r]
        mean = s / jnp.maximum(c, 1.0)[:, None]
        acc = acc + jnp.dot(mean, wl_ref[r], preferred_element_type=jnp.float32)
    out_ref[...] = jnp.maximum(acc, 0.0)


def _tc_combine(sums, cnts, x_job, wl_t, wr_t, bl):
    return pl.pallas_call(
        _tc_body,
        grid=(N // BLK,),
        in_specs=[
            pl.BlockSpec((5 * NC, BLK, D), lambda i: (0, i, 0)),
            pl.BlockSpec((NC, BLK, D), lambda i: (0, i, 0)),
            pl.BlockSpec((BLK, D), lambda i: (i, 0)),
            pl.BlockSpec((5, D, D), lambda i: (0, 0, 0)),
            pl.BlockSpec((D, D), lambda i: (0, 0)),
            pl.BlockSpec((1, D), lambda i: (0, 0)),
        ],
        out_specs=pl.BlockSpec((BLK, D), lambda i: (i, 0)),
        out_shape=jax.ShapeDtypeStruct((N, D), jnp.float32),
    )(sums, cnts, x_job, wl_t, wr_t, bl)


def _prep_edges(ei):
    ei = ei.astype(jnp.int32)
    src = jnp.concatenate([ei[0], jnp.zeros((EPAD - E,), jnp.int32)])
    dst = jnp.concatenate([ei[1], jnp.full((EPAD - E,), N, jnp.int32)])
    src = src.reshape(NW * NCHUNK, CH)
    rep = (jnp.arange(NW * NCHUNK, dtype=jnp.int32) % TREP) * N
    return (src + rep[:, None], dst.reshape(NW * NCHUNK, CH))


def kernel(x_station, x_machine, x_robot, x_job,
           edge_can_load, edge_loaded, edge_will_execute, edge_execute,
           edge_hold,
           Wl_can_load, bl_can_load, Wr_can_load,
           Wl_loaded, bl_loaded, Wr_loaded,
           Wl_will_execute, bl_will_execute, Wr_will_execute,
           Wl_execute, bl_execute, Wr_execute,
           Wl_hold, bl_hold, Wr_hold):
    edges = [_prep_edges(e) for e in (edge_can_load, edge_loaded,
                                      edge_will_execute, edge_execute,
                                      edge_hold)]
    rep = (TREP, 1)
    sums = _sc_features(jnp.tile(x_station, rep), jnp.tile(x_machine, rep),
                        jnp.tile(x_robot, rep), edges)
    cnts = _sc_counts(edges)

    wl_t = jnp.stack([Wl_can_load.T, Wl_loaded.T, Wl_will_execute.T,
                      Wl_execute.T, Wl_hold.T])
    wr_t = (Wr_can_load + Wr_loaded + Wr_will_execute
            + Wr_execute + Wr_hold).T
    bl = (bl_can_load + bl_loaded + bl_will_execute
          + bl_execute + bl_hold).reshape(1, D)
    return _tc_combine(sums, cnts, x_job, wl_t, wr_t, bl)


# speedup vs baseline: 1.0555x; 1.0184x over previous
import functools

import jax
import jax.numpy as jnp
from jax import lax
from jax.experimental import pallas as pl
from jax.experimental.pallas import tpu as pltpu
from jax.experimental.pallas import tpu_sc as plsc

N = 10000
D = 128
E = 320000
NC = 2
NS = 16
NW = NC * NS
CH = 128
NCHUNK = 80
NBUF = 2
IDXG = 8
EPAD = NW * NCHUNK * CH
ACC_ROWS = 10112
ROWS_PER_TILE = ACC_ROWS // NS
CNT_W = 16
BLK = 400
FEAT_SPLIT = (128, 32)
TREP = 32

_MESH = dict(core_axis_name="c", subcore_axis_name="s")


def _sc_features(xs, xm, xr, edges):
    zeros_f = jnp.zeros((CH, D), jnp.float32)

    C0, C1 = FEAT_SPLIT

    @functools.partial(
        pl.kernel,
        out_type=jax.ShapeDtypeStruct((5 * NC, ACC_ROWS, D), jnp.float32),
        mesh=plsc.VectorSubcoreMesh(**_MESH),
        scratch_types=[
            pltpu.VMEM_SHARED((ACC_ROWS, D), jnp.float32),
            pltpu.VMEM((2, IDXG, CH), jnp.int32),
            pltpu.VMEM((2, IDXG, CH), jnp.int32),
            pltpu.VMEM((NBUF, CH, D), jnp.float32),
            pltpu.SemaphoreType.DMA,
            pltpu.SemaphoreType.DMA,
        ],
    )
    def agg(xs_h, xm_h, xr_h,
            s0, d0, s1, d1, s2, d2, s3, d3, s4, d4, zf_h,
            sums_h,
            acc_f, sidx, didx, rows,
            sem0, sem1):
        cid = lax.axis_index("c")
        sid = lax.axis_index("s")
        base = sid * ROWS_PER_TILE
        crow = jnp.where(cid == 0, sid * C0, NS * C0 + sid * C1)
        ngrp = jnp.where(cid == 0, C0 // IDXG, C1 // IDXG)
        sems = [sem0, sem1]

        def wait_buf(tab, b):
            pltpu.make_async_copy(tab.at[pl.ds(0, CH)], rows.at[b],
                                  sems[b]).wait()

        rels = [(xs_h, s0, d0), (xs_h, s1, d1), (xm_h, s2, d2),
                (xm_h, s3, d3), (xr_h, s4, d4)]
        for r, (tab, se, de) in enumerate(rels):
            pltpu.sync_copy(zf_h, rows.at[0])
            for k in range(5):
                n = min(CH, ROWS_PER_TILE - k * CH)
                pltpu.sync_copy(rows.at[0, pl.ds(0, n)],
                                acc_f.at[pl.ds(base + k * CH, n)])
            plsc.subcore_barrier()

            pltpu.sync_copy(se.at[pl.ds(crow, IDXG)], sidx.at[0])
            pltpu.sync_copy(de.at[pl.ds(crow, IDXG)], didx.at[0])
            for b in range(NBUF):
                pltpu.async_copy(tab.at[sidx.at[0, b]], rows.at[b], sems[b])

            def group(g, carry):
                gn = lax.rem(g + 1, ngrp)
                pltpu.sync_copy(se.at[pl.ds(crow + gn * IDXG, IDXG)],
                                sidx.at[(g + 1) % 2])
                pltpu.sync_copy(de.at[pl.ds(crow + gn * IDXG, IDXG)],
                                didx.at[(g + 1) % 2])
                for jj in range(IDXG):
                    b = jj % NBUF
                    wait_buf(tab, b)
                    pltpu.sync_copy(rows.at[b],
                                    acc_f.at[didx.at[g % 2, jj]], add=True)
                    if jj + NBUF < IDXG:
                        pltpu.async_copy(tab.at[sidx.at[g % 2, jj + NBUF]],
                                         rows.at[b], sems[b])
                    else:
                        pltpu.async_copy(
                            tab.at[sidx.at[(g + 1) % 2, jj + NBUF - IDXG]],
                            rows.at[b], sems[b])
                return carry

            lax.fori_loop(0, ngrp, group, 0, unroll=False)
            for b in range(NBUF):
                wait_buf(tab, b)
            plsc.subcore_barrier()

            for k in range(5):
                n = min(CH, ROWS_PER_TILE - k * CH)
                pltpu.sync_copy(acc_f.at[pl.ds(base + k * CH, n)],
                                rows.at[0, pl.ds(0, n)])
                pltpu.sync_copy(rows.at[0, pl.ds(0, n)],
                                sums_h.at[r * NC + cid,
                                          pl.ds(base + k * CH, n)])

    flat = []
    for s, d in edges:
        flat += [s, d]
    return agg(xs, xm, xr, *flat, zeros_f)


def _sc_counts(edges):
    import numpy as np
    pat = np.zeros((5, CH, D), np.float32)
    for r in range(5):
        pat[r, :, 16 * r:16 * (r + 1)] = 1.0
    ones_pat = jnp.asarray(pat)
    zeros_f = jnp.zeros((CH, D), jnp.float32)

    @functools.partial(
        pl.kernel,
        out_type=jax.ShapeDtypeStruct((NC, ACC_ROWS, D), jnp.float32),
        mesh=plsc.VectorSubcoreMesh(**_MESH),
        scratch_types=[
            pltpu.VMEM_SHARED((ACC_ROWS, D), jnp.float32),
            pltpu.VMEM((IDXG, CH), jnp.int32),
            pltpu.VMEM((CH, D), jnp.float32),
            pltpu.VMEM((CH, D), jnp.float32),
        ],
    )
    def cnt(d0, d1, d2, d3, d4, ones_h, zf_h,
            cnts_h,
            acc, didx, ones_v, stage):
        cid = lax.axis_index("c")
        sid = lax.axis_index("s")
        base = sid * ROWS_PER_TILE
        crow = (cid * NS + sid) * NCHUNK

        pltpu.sync_copy(zf_h, stage)
        for k in range(5):
            n = min(CH, ROWS_PER_TILE - k * CH)
            pltpu.sync_copy(stage.at[pl.ds(0, n)],
                            acc.at[pl.ds(base + k * CH, n)])
        plsc.subcore_barrier()

        for r, de in enumerate([d0, d1, d2, d3, d4]):
            pltpu.sync_copy(ones_h.at[r], ones_v)

            def group(g, carry):
                pltpu.sync_copy(de.at[pl.ds(crow + g * IDXG, IDXG)], didx)
                for jj in range(IDXG):
                    pltpu.sync_copy(ones_v, acc.at[didx.at[jj]], add=True)
                return carry

            lax.fori_loop(0, NCHUNK // IDXG, group, 0)
        plsc.subcore_barrier()

        for k in range(5):
            n = min(CH, ROWS_PER_TILE - k * CH)
            pltpu.sync_copy(acc.at[pl.ds(base + k * CH, n)],
                            stage.at[pl.ds(0, n)])
            pltpu.sync_copy(stage.at[pl.ds(0, n)],
                            cnts_h.at[cid, pl.ds(base + k * CH, n)])

    return cnt(*[d for _, d in edges], ones_pat, zeros_f)


def _tc_body(sums_ref, cnts_ref, xj_ref, wl_ref, wr_ref, bl_ref, out_ref):
    acc = jnp.dot(xj_ref[...], wr_ref[...],
                  preferred_element_type=jnp.float32) + bl_ref[...]
    for r in range(5):
        s = sums_ref[2 * r] + sums_ref[2 * r + 1]
        c = cnts_ref[0, :, 16 * r] + cnts_ref[1, :, 16 * r]
        mean = s / jnp.maximum(c, 1.0)[:, None]
        acc = acc + jnp.dot(mean, wl_ref[r], preferred_element_type=jnp.float32)
    out_ref[...] = jnp.maximum(acc, 0.0)


def _tc_combine(sums, cnts, x_job, wl_t, wr_t, bl):
    return pl.pallas_call(
        _tc_body,
        grid=(N // BLK,),
        in_specs=[
            pl.BlockSpec((5 * NC, BLK, D), lambda i: (0, i, 0)),
            pl.BlockSpec((NC, BLK, D), lambda i: (0, i, 0)),
            pl.BlockSpec((BLK, D), lambda i: (i, 0)),
            pl.BlockSpec((5, D, D), lambda i: (0, 0, 0)),
            pl.BlockSpec((D, D), lambda i: (0, 0)),
            pl.BlockSpec((1, D), lambda i: (0, 0)),
        ],
        out_specs=pl.BlockSpec((BLK, D), lambda i: (i, 0)),
        out_shape=jax.ShapeDtypeStruct((N, D), jnp.float32),
    )(sums, cnts, x_job, wl_t, wr_t, bl)


def _prep_edges(ei):
    ei = ei.astype(jnp.int32)
    src = jnp.concatenate([ei[0], jnp.zeros((EPAD - E,), jnp.int32)])
    dst = jnp.concatenate([ei[1], jnp.full((EPAD - E,), N, jnp.int32)])
    src = src.reshape(NW * NCHUNK, CH)
    rep = (jnp.arange(NW * NCHUNK, dtype=jnp.int32) % TREP) * N
    return (src + rep[:, None], dst.reshape(NW * NCHUNK, CH))


def kernel(x_station, x_machine, x_robot, x_job,
           edge_can_load, edge_loaded, edge_will_execute, edge_execute,
           edge_hold,
           Wl_can_load, bl_can_load, Wr_can_load,
           Wl_loaded, bl_loaded, Wr_loaded,
           Wl_will_execute, bl_will_execute, Wr_will_execute,
           Wl_execute, bl_execute, Wr_execute,
           Wl_hold, bl_hold, Wr_hold):
    edges = [_prep_edges(e) for e in (edge_can_load, edge_loaded,
                                      edge_will_execute, edge_execute,
                                      edge_hold)]
    rep = (TREP, 1)
    sums = _sc_features(jnp.tile(x_station, rep), jnp.tile(x_machine, rep),
                        jnp.tile(x_robot, rep), edges)
    cnts = _sc_counts(edges)

    wl_t = jnp.stack([Wl_can_load.T, Wl_loaded.T, Wl_will_execute.T,
                      Wl_execute.T, Wl_hold.T])
    wr_t = (Wr_can_load + Wr_loaded + Wr_will_execute
            + Wr_execute + Wr_hold).T
    bl = (bl_can_load + bl_loaded + bl_will_execute
          + bl_execute + bl_hold).reshape(1, D)
    return _tc_combine(sums, cnts, x_job, wl_t, wr_t, bl)
